# baseline (speedup 1.0000x reference)
import functools

import jax
import jax.numpy as jnp
from jax import lax
from jax.experimental import pallas as pl
from jax.experimental.pallas import tpu as pltpu
from jax.experimental.pallas import tpu_sc as plsc

BATCH = 16384

_INFO = plsc.get_sparse_core_info()
NC = _INFO.num_cores
NS = _INFO.num_subcores
L = _INFO.num_lanes
NW = NC * NS
BPW = BATCH // NW
CHUNK = 128
NCH = BPW // CHUNK

_mesh = plsc.VectorSubcoreMesh(core_axis_name="c", subcore_axis_name="s")


@functools.partial(
    pl.kernel,
    mesh=_mesh,
    compiler_params=pltpu.CompilerParams(use_tc_tiling_on_sc=False),
    out_type=jax.ShapeDtypeStruct((BATCH, 1), jnp.float32),
    scratch_types=[
        pltpu.VMEM((NCH, CHUNK), jnp.int32),
        pltpu.VMEM((NCH, CHUNK), jnp.int32),
        pltpu.VMEM((BPW, 2), jnp.float32),
        pltpu.VMEM((BPW, 2), jnp.float32),
        pltpu.VMEM((BPW, 1), jnp.float32),
        pltpu.VMEM((BPW, 1), jnp.float32),
        pltpu.SemaphoreType.DMA,
    ],
)
def _mirt_sc(stu_hbm, exer_hbm, theta_hbm, alpha_hbm, beta_hbm, out_hbm,
             idx_s, idx_e, th2, al2, be2, out_v, sem):
    wid = lax.axis_index("s") * NC + lax.axis_index("c")
    base = wid * BPW

    for j in range(NCH):
        pltpu.sync_copy(stu_hbm.at[pl.ds(base + j * CHUNK, CHUNK)], idx_s.at[j])
        pltpu.sync_copy(exer_hbm.at[pl.ds(base + j * CHUNK, CHUNK)], idx_e.at[j])

    copies = []
    for j in range(NCH):
        sl = pl.ds(j * CHUNK, CHUNK)
        copies.append(pltpu.async_copy(theta_hbm.at[idx_s.at[j]], th2.at[sl], sem))
        copies.append(pltpu.async_copy(alpha_hbm.at[idx_e.at[j]], al2.at[sl], sem))
        copies.append(pltpu.async_copy(beta_hbm.at[idx_e.at[j]], be2.at[sl], sem))
    for c in copies:
        c.wait()

    for g in range(BPW // L):
        t = th2[pl.ds(g * L, L), :]
        a = al2[pl.ds(g * L, L), :]
        b = be2[pl.ds(g * L, L), :]
        p = a * (t - b)
        pred = p[:, 0:1] + p[:, 1:2]
        out_v[pl.ds(g * L, L), :] = 1.0 / (1.0 + jnp.exp(-pred))

    pltpu.sync_copy(out_v, out_hbm.at[pl.ds(base, BPW)])


def kernel(stu_id, exer_id, theta_table, alpha_table, beta_table):
    out = _mirt_sc(
        stu_id.astype(jnp.int32),
        exer_id.astype(jnp.int32),
        theta_table,
        alpha_table,
        beta_table,
    )
    return jnp.squeeze(out, axis=1)


# speedup vs baseline: 23.4977x; 23.4977x over previous
import functools

import jax
import jax.numpy as jnp
from jax import lax
from jax.experimental import pallas as pl
from jax.experimental.pallas import tpu as pltpu
from jax.experimental.pallas import tpu_sc as plsc

BATCH = 16384

_INFO = plsc.get_sparse_core_info()
NC = _INFO.num_cores
NS = _INFO.num_subcores
L = _INFO.num_lanes
NW = NC * NS
BPW = BATCH // NW

_mesh = plsc.VectorSubcoreMesh(core_axis_name="c", subcore_axis_name="s")


@functools.partial(
    pl.kernel,
    mesh=_mesh,
    out_type=jax.ShapeDtypeStruct((BATCH,), jnp.float32),
    scratch_types=[
        pltpu.VMEM((BPW,), jnp.int32),
        pltpu.VMEM((BPW,), jnp.int32),
        pltpu.VMEM((BPW,), jnp.float32),
        pltpu.VMEM((BPW,), jnp.float32),
        pltpu.VMEM((BPW,), jnp.float32),
        pltpu.VMEM((BPW,), jnp.float32),
        pltpu.VMEM((BPW,), jnp.float32),
        pltpu.VMEM((BPW,), jnp.float32),
        pltpu.SemaphoreType.DMA,
    ],
)
def _mirt_sc(stu_hbm, exer_hbm, t0_hbm, t1_hbm, a0_hbm, a1_hbm, be_hbm,
             out_hbm, idx_s, idx_e, t0_v, t1_v, a0_v, a1_v, be_v, out_v, sem):
    wid = lax.axis_index("s") * NC + lax.axis_index("c")
    base = wid * BPW

    pltpu.sync_copy(stu_hbm.at[pl.ds(base, BPW)], idx_s)
    pltpu.sync_copy(exer_hbm.at[pl.ds(base, BPW)], idx_e)

    copies = [
        pltpu.async_copy(t0_hbm.at[idx_s], t0_v, sem),
        pltpu.async_copy(t1_hbm.at[idx_s], t1_v, sem),
        pltpu.async_copy(a0_hbm.at[idx_e], a0_v, sem),
        pltpu.async_copy(a1_hbm.at[idx_e], a1_v, sem),
        pltpu.async_copy(be_hbm.at[idx_e], be_v, sem),
    ]
    for c in copies:
        c.wait()

    for g in range(BPW // L):
        sl = pl.ds(g * L, L)
        t0 = t0_v[sl]
        t1 = t1_v[sl]
        a0 = a0_v[sl]
        a1 = a1_v[sl]
        b = be_v[sl]
        pred = a0 * (t0 - b) + a1 * (t1 - b)
        out_v[sl] = 1.0 / (1.0 + jnp.exp(-pred))

    pltpu.sync_copy(out_v, out_hbm.at[pl.ds(base, BPW)])


def kernel(stu_id, exer_id, theta_table, alpha_table, beta_table):
    return _mirt_sc(
        stu_id.astype(jnp.int32),
        exer_id.astype(jnp.int32),
        theta_table[:, 0],
        theta_table[:, 1],
        alpha_table[:, 0],
        alpha_table[:, 1],
        jnp.reshape(beta_table, (-1,)),
    )
